# Initial kernel scaffold; baseline (speedup 1.0000x reference)
import jax
import jax.numpy as jnp
from jax.experimental import pallas as pl


def kernel(x, edge_index, edge_attr, batch, train, eps0, c0_W1, c0_b1, c0_g1, c0_be1, c0_W2, c0_b2, bn0_g, bn0_b, eps1, c1_W1, c1_b1, c1_g1, c1_be1, c1_W2, c1_b2, bn1_g, bn1_b, vn_W1, vn_b1, vn_g1, vn_be1, vn_W2, vn_b2, vn_g2, vn_be2, pred_W, pred_b):
    raise NotImplementedError("write your pallas kernel here")


# speedup vs baseline: 4.6555x; 4.6555x over previous
import functools

import jax
import jax.numpy as jnp
from jax import lax
from jax.experimental import pallas as pl
from jax.experimental.pallas import tpu as pltpu
from jax.experimental.pallas import tpu_sc as plsc

N = 10000
E = 320000
D = 128
H = 256
G = 128

NC = 2
NS = 16
NW = NC * NS
CH = 80
EPT = E // NW
NCHUNK = EPT // CH
NPAD = 10240
RPS = NPAD // NS


SUP = 2000
NSUP = EPT // SUP
CPS = SUP // CH


def _edge_body(h_hbm, src_hbm, dst_hbm, ea_hbm, zero_hbm, out_hbm,
               src_sv, dst_sv, dst_v, lin_v, hrow_v, ea_v, acc_sh,
               sem_g, sem_e):
    cid = lax.axis_index("c")
    sid = lax.axis_index("s")
    wid = cid * NS + sid
    base = wid * EPT

    pltpu.sync_copy(zero_hbm.at[pl.ds(sid * RPS, RPS)],
                    acc_sh.at[pl.ds(sid * RPS, RPS)])
    plsc.subcore_barrier()

    iota16 = lax.iota(jnp.int32, 16)

    def sup(si, carry):
        soff = base + si * SUP
        pltpu.sync_copy(src_hbm.at[pl.ds(soff, SUP)], src_sv)
        pltpu.sync_copy(dst_hbm.at[pl.ds(soff, SUP)], dst_sv)

        def chunk(k, c1):
            off = soff + k * CH
            for c in range(CH // 16):
                lin_v[pl.ds(c * 16, 16)] = iota16 + (off + c * 16)
                dst_v[pl.ds(c * 16, 16)] = dst_sv[pl.ds(k * CH + c * 16, 16)]
            cp_g = pltpu.async_copy(
                h_hbm.at[src_sv.at[pl.ds(k * CH, CH)]], hrow_v, sem_g)
            cp_e = pltpu.async_copy(ea_hbm.at[lin_v], ea_v, sem_e)
            cp_g.wait()
            cp_e.wait()

            def row(r, c2):
                for c in range(D // 16):
                    sl = pl.ds(c * 16, 16)
                    v = hrow_v[r, sl] + ea_v[r, sl]
                    hrow_v[r, sl] = jnp.maximum(v, 0.0)
                return c2

            lax.fori_loop(0, CH, row, 0)
            pltpu.sync_copy(hrow_v, acc_sh.at[dst_v], add=True)
            return c1

        lax.fori_loop(0, CPS, chunk, 0)
        return carry

    lax.fori_loop(0, NSUP, sup, 0)
    plsc.subcore_barrier()
    pltpu.sync_copy(acc_sh.at[pl.ds(sid * RPS, RPS)],
                    out_hbm.at[cid].at[pl.ds(sid * RPS, RPS)])


@functools.lru_cache(maxsize=None)
def _edge_kernel():
    return pl.kernel(
        _edge_body,
        out_type=jax.ShapeDtypeStruct((NC, NPAD, D), jnp.float32),
        mesh=plsc.VectorSubcoreMesh(core_axis_name="c", subcore_axis_name="s",
                                    num_cores=NC, num_subcores=NS),
        scratch_types=[
            pltpu.VMEM((SUP,), jnp.int32),
            pltpu.VMEM((SUP,), jnp.int32),
            pltpu.VMEM((CH,), jnp.int32),
            pltpu.VMEM((CH,), jnp.int32),
            pltpu.VMEM((CH, D), jnp.float32),
            pltpu.VMEM((CH, D), jnp.float32),
            pltpu.VMEM_SHARED((NPAD, D), jnp.float32),
            pltpu.SemaphoreType.DMA,
            pltpu.SemaphoreType.DMA,
        ],
    )


def _edge_call(h, src, dst, ea, zeros):
    return _edge_kernel()(h, src, dst, ea, zeros)


def _bn(z, g, b):
    mu = jnp.mean(z, axis=0, keepdims=True)
    var = jnp.mean((z - mu) ** 2, axis=0, keepdims=True)
    return (z - mu) / jnp.sqrt(var + 1e-05) * g + b


def _mm(a, b):
    return jnp.dot(a, b, preferred_element_type=jnp.float32)


def _mmh(a, b):
    return jnp.dot(a, b, preferred_element_type=jnp.float32,
                   precision=lax.Precision.HIGHEST)


def _mmT(a, b):
    return lax.dot_general(a, b, (((0,), (0,)), ((), ())),
                           preferred_element_type=jnp.float32,
                           precision=lax.Precision.HIGHEST)


def _dense0_body(h0_r, agg_r, batch_r, eps_r,
                 W1_r, b1_r, g1_r, be1_r, W2_r, b2_r, bng_r, bnb_r,
                 vW1_r, vb1_r, vg1_r, vbe1_r, vW2_r, vb2_r, vg2_r, vbe2_r,
                 out_r):
    h0 = h0_r[...]
    pre = (1.0 + eps_r[0, 0]) * h0 + agg_r[0, :N] + agg_r[1, :N]
    z = _bn(_mm(pre, W1_r[...]) + b1_r[...], g1_r[...], be1_r[...])
    m = jnp.maximum(z, 0.0)
    h = _mm(m, W2_r[...]) + b2_r[...]
    h = _bn(h, bng_r[...], bnb_r[...])
    h1a = jnp.maximum(h, 0.0)
    oh = (batch_r[...] == lax.broadcasted_iota(jnp.int32, (N, G), 1))
    oh = oh.astype(jnp.float32)
    vt = _mmT(oh, h0)
    zz = _bn(_mm(vt, vW1_r[...]) + vb1_r[...], vg1_r[...], vbe1_r[...])
    vmid = jnp.maximum(zz, 0.0)
    zz2 = _bn(_mm(vmid, vW2_r[...]) + vb2_r[...], vg2_r[...], vbe2_r[...])
    vn = jnp.maximum(zz2, 0.0)
    out_r[...] = h1a + _mmh(oh, vn)


def _dense1_body(h1_r, agg_r, batch_r, eps_r,
                 W1_r, b1_r, g1_r, be1_r, W2_r, b2_r, bng_r, bnb_r,
                 pW_r, pb_r, out_r):
    h1 = h1_r[...]
    pre = (1.0 + eps_r[0, 0]) * h1 + agg_r[0, :N] + agg_r[1, :N]
    z = _bn(_mm(pre, W1_r[...]) + b1_r[...], g1_r[...], be1_r[...])
    m = jnp.maximum(z, 0.0)
    h = _mm(m, W2_r[...]) + b2_r[...]
    hn = _bn(h, bng_r[...], bnb_r[...])
    oh = (batch_r[...] == lax.broadcasted_iota(jnp.int32, (N, G), 1))
    oh = oh.astype(jnp.float32)
    sums = _mmT(oh, hn)
    counts = _mmT(oh, jnp.ones((N, 1), jnp.float32))
    h_graph = sums / jnp.maximum(counts, 1.0)
    out_r[...] = _mm(h_graph, pW_r[...]) + pb_r[...]


_TC_PARAMS = pltpu.CompilerParams(vmem_limit_bytes=100 * 1024 * 1024)
_dense0 = pl.pallas_call(
    _dense0_body, out_shape=jax.ShapeDtypeStruct((N, D), jnp.float32),
    compiler_params=_TC_PARAMS)
_dense1 = pl.pallas_call(
    _dense1_body, out_shape=jax.ShapeDtypeStruct((G, 1), jnp.float32),
    compiler_params=_TC_PARAMS)


def kernel(x, edge_index, edge_attr, batch, train, eps0, c0_W1, c0_b1,
           c0_g1, c0_be1, c0_W2, c0_b2, bn0_g, bn0_b, eps1, c1_W1, c1_b1,
           c1_g1, c1_be1, c1_W2, c1_b2, bn1_g, bn1_b, vn_W1, vn_b1, vn_g1,
           vn_be1, vn_W2, vn_b2, vn_g2, vn_be2, pred_W, pred_b):
    src = edge_index[0]
    dst = edge_index[1]
    zeros = jnp.zeros((NPAD, D), jnp.float32)
    batch2 = batch.reshape(N, 1)
    r1 = lambda v: v.reshape(1, -1)

    h0 = x
    agg0 = _edge_call(h0, src, dst, edge_attr, zeros)
    h1 = _dense0(h0, agg0, batch2, eps0.reshape(1, 1),
                 c0_W1, r1(c0_b1), r1(c0_g1), r1(c0_be1), c0_W2, r1(c0_b2),
                 r1(bn0_g), r1(bn0_b),
                 vn_W1, r1(vn_b1), r1(vn_g1), r1(vn_be1), vn_W2, r1(vn_b2),
                 r1(vn_g2), r1(vn_be2))
    agg1 = _edge_call(h1, src, dst, edge_attr, zeros)
    return _dense1(h1, agg1, batch2, eps1.reshape(1, 1),
                   c1_W1, r1(c1_b1), r1(c1_g1), r1(c1_be1), c1_W2,
                   r1(c1_b2), r1(bn1_g), r1(bn1_b), pred_W, r1(pred_b))
